# baseline (speedup 1.0000x reference)
import functools
import math

import jax
import jax.numpy as jnp
from jax import lax
from jax.experimental import pallas as pl
from jax.experimental.pallas import tpu as pltpu
from jax.experimental.pallas import tpu_sc as plsc

D_MODEL = 1024
SCALE = math.sqrt(D_MODEL)
NUM_CORES = 2
NUM_SUBCORES = 16
NW = NUM_CORES * NUM_SUBCORES
LANES = 16
CHUNK = 16
NBUF = 4
SW = 2


@functools.lru_cache(maxsize=None)
def _make_sc_kernel(B):
    assert B % (8 * NW) == 0
    bpw = B // NW
    nch = bpw // CHUNK
    assert nch % NBUF == 0 and nch // NBUF >= 3
    assert 1 <= SW <= NBUF - 2
    ngrp = nch // NBUF
    mesh = plsc.VectorSubcoreMesh(core_axis_name="c", subcore_axis_name="s")

    @functools.partial(
        pl.kernel,
        mesh=mesh,
        out_type=jax.ShapeDtypeStruct((B, D_MODEL), jnp.float32),
        scratch_types=[
            pltpu.VMEM((bpw,), jnp.int32),
        ]
        + [pltpu.VMEM((CHUNK, D_MODEL), jnp.float32) for _ in range(NBUF)]
        + [pltpu.SemaphoreType.DMA for _ in range(2 * NBUF)],
    )
    def emb_kernel(table_hbm, idx_hbm, out_hbm, idx_v, *rest):
        bufs = rest[:NBUF]
        gsem = rest[NBUF : 2 * NBUF]
        ssem = rest[2 * NBUF :]
        wid = lax.axis_index("s") * NUM_CORES + lax.axis_index("c")
        base = wid * bpw
        pltpu.sync_copy(idx_hbm.at[pl.ds(base, bpw)], idx_v)

        def gather_copy(ci, b):
            return pltpu.make_async_copy(
                table_hbm.at[idx_v.at[pl.ds(ci * CHUNK, CHUNK)]], bufs[b], gsem[b]
            )

        def store_copy(ci, b):
            return pltpu.make_async_copy(
                bufs[b], out_hbm.at[pl.ds(base + ci * CHUNK, CHUNK)], ssem[b]
            )

        def scale(b):
            def row(r, c):
                for v in range(D_MODEL // LANES):
                    sl = pl.ds(v * LANES, LANES)
                    bufs[b][r, sl] = bufs[b][r, sl] * SCALE
                return c

            lax.fori_loop(0, CHUNK, row, 0)

        def step(ci, b, do_store_wait, do_gather_issue):
            bp = (b + NBUF - SW) % NBUF
            gather_copy(ci, b).wait()
            scale(b)
            store_copy(ci, b).start()
            if do_store_wait:
                store_copy(ci - SW, bp).wait()
            if do_gather_issue:
                gather_copy(ci + NBUF - SW, bp).start()

        for b in range(NBUF):
            gather_copy(b, b).start()

        for b in range(NBUF):
            step(b, b, b >= SW, b >= SW)

        def group(g, c):
            for b in range(NBUF):
                step(g * NBUF + b, b, True, True)
            return c

        lax.fori_loop(1, ngrp - 1, group, 0)

        ci0 = (ngrp - 1) * NBUF
        for b in range(NBUF):
            step(ci0 + b, b, True, b < SW)
        for k in range(SW):
            ci = nch - SW + k
            store_copy(ci, ci % NBUF).wait()

    return emb_kernel


def kernel(x, embedding):
    idx = x.reshape(-1).astype(jnp.int32)
    out = _make_sc_kernel(idx.shape[0])(embedding, idx)
    return out.reshape(x.shape + (D_MODEL,))


# speedup vs baseline: 1.0529x; 1.0529x over previous
import functools
import math

import jax
import jax.numpy as jnp
from jax import lax
from jax.experimental import pallas as pl
from jax.experimental.pallas import tpu as pltpu
from jax.experimental.pallas import tpu_sc as plsc

D_MODEL = 1024
SCALE = math.sqrt(D_MODEL)
NUM_CORES = 2
NUM_SUBCORES = 16
NW = NUM_CORES * NUM_SUBCORES
LANES = 16
CHUNK = 16
NBUF = 4
SW = 1


@functools.lru_cache(maxsize=None)
def _make_sc_kernel(BS, SEQ):
    B = BS * SEQ
    assert B % (8 * NW) == 0
    bpw = B // NW
    assert SEQ % bpw == 0
    wps = SEQ // bpw
    nch = bpw // CHUNK
    assert nch % NBUF == 0 and nch // NBUF >= 3
    assert 1 <= SW <= NBUF - 2
    ngrp = nch // NBUF
    mesh = plsc.VectorSubcoreMesh(core_axis_name="c", subcore_axis_name="s")

    @functools.partial(
        pl.kernel,
        mesh=mesh,
        out_type=jax.ShapeDtypeStruct((BS, SEQ, D_MODEL), jnp.float32),
        scratch_types=[
            pltpu.VMEM((bpw,), jnp.int32),
        ]
        + [pltpu.VMEM((CHUNK, D_MODEL), jnp.float32) for _ in range(NBUF)]
        + [pltpu.SemaphoreType.DMA for _ in range(2 * NBUF)],
    )
    def emb_kernel(table_hbm, idx_hbm, out_hbm, idx_v, *rest):
        bufs = rest[:NBUF]
        gsem = rest[NBUF : 2 * NBUF]
        ssem = rest[2 * NBUF :]
        wid = lax.axis_index("s") * NUM_CORES + lax.axis_index("c")
        seq_i = wid // wps
        col0 = (wid % wps) * bpw
        pltpu.sync_copy(idx_hbm.at[seq_i, pl.ds(col0, bpw)], idx_v)

        def gather_copy(ci, b):
            return pltpu.make_async_copy(
                table_hbm.at[idx_v.at[pl.ds(ci * CHUNK, CHUNK)]], bufs[b], gsem[b]
            )

        def store_copy(ci, b):
            return pltpu.make_async_copy(
                bufs[b], out_hbm.at[seq_i, pl.ds(col0 + ci * CHUNK, CHUNK)], ssem[b]
            )

        def scale(b):
            def row(r, c):
                for v in range(D_MODEL // LANES):
                    sl = pl.ds(v * LANES, LANES)
                    bufs[b][r, sl] = bufs[b][r, sl] * SCALE
                return c

            lax.fori_loop(0, CHUNK, row, 0)

        def step(ci, b, do_store_wait, do_gather_issue):
            bp = (b + NBUF - SW) % NBUF
            gather_copy(ci, b).wait()
            scale(b)
            store_copy(ci, b).start()
            if do_store_wait:
                store_copy(ci - SW, bp).wait()
            if do_gather_issue:
                gather_copy(ci + NBUF - SW, bp).start()

        for b in range(NBUF):
            gather_copy(b, b).start()

        for b in range(NBUF):
            step(b, b, b >= SW, b >= SW)

        def group(g, c):
            for b in range(NBUF):
                step(g * NBUF + b, b, True, True)
            return c

        lax.fori_loop(1, ngrp - 1, group, 0)

        ci0 = (ngrp - 1) * NBUF
        for b in range(NBUF):
            step(ci0 + b, b, True, b < SW)
        for k in range(SW):
            ci = nch - SW + k
            store_copy(ci, ci % NBUF).wait()

    return emb_kernel


def kernel(x, embedding):
    idx = x.astype(jnp.int32)
    return _make_sc_kernel(x.shape[0], x.shape[1])(embedding, idx)
